# Initial kernel scaffold; baseline (speedup 1.0000x reference)
import jax
import jax.numpy as jnp
from jax.experimental import pallas as pl


def kernel(x, f, y_i, y_s, n_true, e_true, l_p):
    raise NotImplementedError("write your pallas kernel here")


# speedup vs baseline: 1.7972x; 1.7972x over previous
import jax
import jax.numpy as jnp
from jax.experimental import pallas as pl

N_HIT = 50000
N_TRUE = 256
N_EDGE = 30000
S_B = 1.0
Q_MIN = 0.5
E_B = 2000
H_B = 2000


def _atanh2(v):
    a = 0.5 * jnp.log((1.0 + v) / (1.0 - v))
    return a * a


def _segmax_kernel(ep_ref, vals_ref, out_ref):
    i = pl.program_id(0)

    @pl.when(i == 0)
    def _():
        out_ref[...] = jnp.full(out_ref.shape, -1.0, jnp.float32)

    lane = jax.lax.broadcasted_iota(jnp.int32, (E_B, N_TRUE), 1)
    mask = ep_ref[...] == lane
    contrib = jnp.where(mask, vals_ref[...], -1.0)
    out_ref[...] = jnp.maximum(out_ref[...],
                               jnp.max(contrib, axis=0, keepdims=True))


def _argmin_kernel(ep_ref, vals_ref, segmax_ref, out_ref):
    i = pl.program_id(0)

    @pl.when(i == 0)
    def _():
        out_ref[...] = jnp.full(out_ref.shape, N_EDGE, jnp.int32)

    lane = jax.lax.broadcasted_iota(jnp.int32, (E_B, N_TRUE), 1)
    row = jax.lax.broadcasted_iota(jnp.int32, (E_B, 1), 0)
    eidx = i * E_B + row
    cond = (ep_ref[...] == lane) & (vals_ref[...] >= segmax_ref[...])
    cand = jnp.where(cond, eidx, N_EDGE)
    out_ref[...] = jnp.minimum(out_ref[...],
                               jnp.min(cand, axis=0, keepdims=True))


def _hit_kernel(x_ref, f_ref, yi_ref, ys_ref, lp_ref, xct_ref, fc_ref,
                segmax_ref, out_ref):
    i = pl.program_id(0)

    @pl.when(i == 0)
    def _():
        out_ref[...] = jnp.zeros(out_ref.shape, jnp.float32)

    f = f_ref[...]
    q = _atanh2(f) + Q_MIN
    qc = _atanh2(fc_ref[...]) + Q_MIN

    dist = jnp.zeros((H_B, N_TRUE), jnp.float32)
    for d in range(3):
        diff = x_ref[:, d:d + 1] - xct_ref[d:d + 1, :]
        dist = dist + diff * diff
    base = jnp.maximum(1.0 - dist, 0.0) * qc
    v_base = jnp.sum(jnp.sum(base, axis=1, keepdims=True) * q)

    bkg = (yi_ref[...] == -1) & (ys_ref[...] >= 0)
    bkgf = bkg.astype(jnp.float32)
    n_bkg = jnp.sum(bkgf)
    bkg_sum = jnp.sum(bkgf * f)
    xi = jnp.where(bkg, 0.0, _atanh2(f))
    xi_sum = jnp.sum(xi)
    lp_xi = jnp.sum(jnp.sum(lp_ref[...], axis=1, keepdims=True) * xi)
    fcs = jnp.where(i == 0,
                    jnp.sum(jnp.maximum(segmax_ref[...], 0.0)), 0.0)

    lane = jax.lax.broadcasted_iota(jnp.int32, (1, 128), 1)
    upd = (jnp.where(lane == 0, v_base, 0.0)
           + jnp.where(lane == 1, n_bkg, 0.0)
           + jnp.where(lane == 2, bkg_sum, 0.0)
           + jnp.where(lane == 3, xi_sum, 0.0)
           + jnp.where(lane == 4, lp_xi, 0.0)
           + jnp.where(lane == 5, fcs, 0.0))
    out_ref[...] = out_ref[...] + upd


def _edge_kernel(ep_ref, xe_ref, fe_ref, w_ref, xct_ref, fc_ref, out_ref):
    i = pl.program_id(0)

    @pl.when(i == 0)
    def _():
        out_ref[...] = jnp.zeros(out_ref.shape, jnp.float32)

    lane = jax.lax.broadcasted_iota(jnp.int32, (E_B, N_TRUE), 1)
    mask = ep_ref[...] == lane
    dist = jnp.zeros((E_B, N_TRUE), jnp.float32)
    for d in range(3):
        diff = xe_ref[:, d:d + 1] - xct_ref[d:d + 1, :]
        dist = dist + diff * diff
    d_e = jnp.sum(jnp.where(mask, dist, 0.0), axis=1, keepdims=True)
    qc = _atanh2(fc_ref[...]) + Q_MIN
    qcg = jnp.sum(jnp.where(mask, qc, 0.0), axis=1, keepdims=True)
    qe = _atanh2(fe_ref[...]) + Q_MIN
    corr = jnp.sum(w_ref[...] * qe * qcg
                   * (d_e - jnp.maximum(1.0 - d_e, 0.0)))
    lane128 = jax.lax.broadcasted_iota(jnp.int32, (1, 128), 1)
    out_ref[...] = out_ref[...] + jnp.where(lane128 == 0, corr, 0.0)


def kernel(x, f, y_i, y_s, n_true, e_true, l_p):
    x = x.astype(jnp.float32)
    f = f.astype(jnp.float32)
    e_h = e_true[0].astype(jnp.int32)
    e_p = e_true[1].astype(jnp.int32)
    vals = f[e_h]

    ep2 = e_p.reshape(-1, 1)
    vals2 = vals.reshape(-1, 1)
    n_eb = N_EDGE // E_B

    segmax = pl.pallas_call(
        _segmax_kernel,
        grid=(n_eb,),
        in_specs=[pl.BlockSpec((E_B, 1), lambda i: (i, 0)),
                  pl.BlockSpec((E_B, 1), lambda i: (i, 0))],
        out_specs=pl.BlockSpec((1, N_TRUE), lambda i: (0, 0)),
        out_shape=jax.ShapeDtypeStruct((1, N_TRUE), jnp.float32),
    )(ep2, vals2)

    arg = pl.pallas_call(
        _argmin_kernel,
        grid=(n_eb,),
        in_specs=[pl.BlockSpec((E_B, 1), lambda i: (i, 0)),
                  pl.BlockSpec((E_B, 1), lambda i: (i, 0)),
                  pl.BlockSpec((1, N_TRUE), lambda i: (0, 0))],
        out_specs=pl.BlockSpec((1, N_TRUE), lambda i: (0, 0)),
        out_shape=jax.ShapeDtypeStruct((1, N_TRUE), jnp.int32),
    )(ep2, vals2, segmax)

    centers = e_h[arg[0]]
    xct = x[centers].T
    fc_row = f[centers].reshape(1, N_TRUE)

    keys = jnp.sort(e_h * N_TRUE + e_p)
    prev = jnp.concatenate([jnp.full((1,), -1, jnp.int32), keys[:-1]])
    w = (keys != prev).astype(jnp.float32).reshape(-1, 1)
    eh_s = keys // N_TRUE
    ep_s = (keys % N_TRUE).reshape(-1, 1)
    xe = x[eh_s]
    fe = f[eh_s].reshape(-1, 1)

    n_hb = N_HIT // H_B
    acc = pl.pallas_call(
        _hit_kernel,
        grid=(n_hb,),
        in_specs=[pl.BlockSpec((H_B, 3), lambda i: (i, 0)),
                  pl.BlockSpec((H_B, 1), lambda i: (i, 0)),
                  pl.BlockSpec((H_B, 1), lambda i: (i, 0)),
                  pl.BlockSpec((H_B, 1), lambda i: (i, 0)),
                  pl.BlockSpec((H_B, 4), lambda i: (i, 0)),
                  pl.BlockSpec((3, N_TRUE), lambda i: (0, 0)),
                  pl.BlockSpec((1, N_TRUE), lambda i: (0, 0)),
                  pl.BlockSpec((1, N_TRUE), lambda i: (0, 0))],
        out_specs=pl.BlockSpec((1, 128), lambda i: (0, 0)),
        out_shape=jax.ShapeDtypeStruct((1, 128), jnp.float32),
    )(x, f.reshape(-1, 1), y_i.reshape(-1, 1).astype(jnp.int32),
      y_s.reshape(-1, 1).astype(jnp.int32), l_p.astype(jnp.float32),
      xct, fc_row, segmax)

    corr = pl.pallas_call(
        _edge_kernel,
        grid=(n_eb,),
        in_specs=[pl.BlockSpec((E_B, 1), lambda i: (i, 0)),
                  pl.BlockSpec((E_B, 3), lambda i: (i, 0)),
                  pl.BlockSpec((E_B, 1), lambda i: (i, 0)),
                  pl.BlockSpec((E_B, 1), lambda i: (i, 0)),
                  pl.BlockSpec((3, N_TRUE), lambda i: (0, 0)),
                  pl.BlockSpec((1, N_TRUE), lambda i: (0, 0))],
        out_specs=pl.BlockSpec((1, 128), lambda i: (0, 0)),
        out_shape=jax.ShapeDtypeStruct((1, 128), jnp.float32),
    )(ep_s, xe, fe, w, xct, fc_row)

    v_base = acc[0, 0]
    n_bkg = acc[0, 1]
    bkg_sum = acc[0, 2]
    xi_sum = acc[0, 3]
    lp_xi = acc[0, 4]
    fc_sum = acc[0, 5]

    b = 1.0 - fc_sum / n_true
    b = b + jnp.where(n_bkg > 0,
                      (S_B / jnp.maximum(n_bkg, 1.0)) * bkg_sum, 0.0)
    v = (v_base + corr[0, 0]) / N_HIT
    p = lp_xi / xi_sum
    return jnp.stack([b, v, p])


# speedup vs baseline: 1.8453x; 1.0268x over previous
import jax
import jax.numpy as jnp
from jax.experimental import pallas as pl

N_HIT = 50000
N_TRUE = 256
N_EDGE = 30000
S_B = 1.0
Q_MIN = 0.5
E_B = 3000
H_B = 5000


def _atanh2(v):
    a = 0.5 * jnp.log((1.0 + v) / (1.0 - v))
    return a * a


def _segmax_kernel(ep_ref, vals_ref, out_ref):
    i = pl.program_id(0)

    @pl.when(i == 0)
    def _():
        out_ref[...] = jnp.full(out_ref.shape, -1.0, jnp.float32)

    lane = jax.lax.broadcasted_iota(jnp.int32, (E_B, N_TRUE), 1)
    mask = ep_ref[...] == lane
    contrib = jnp.where(mask, vals_ref[...], -1.0)
    out_ref[...] = jnp.maximum(out_ref[...],
                               jnp.max(contrib, axis=0, keepdims=True))


def _argmin_kernel(ep_ref, vals_ref, segmax_ref, out_ref):
    i = pl.program_id(0)

    @pl.when(i == 0)
    def _():
        out_ref[...] = jnp.full(out_ref.shape, N_EDGE, jnp.int32)

    lane = jax.lax.broadcasted_iota(jnp.int32, (E_B, N_TRUE), 1)
    row = jax.lax.broadcasted_iota(jnp.int32, (E_B, 1), 0)
    eidx = i * E_B + row
    cond = (ep_ref[...] == lane) & (vals_ref[...] >= segmax_ref[...])
    cand = jnp.where(cond, eidx, N_EDGE)
    out_ref[...] = jnp.minimum(out_ref[...],
                               jnp.min(cand, axis=0, keepdims=True))


def _hit_kernel(x_ref, f_ref, yi_ref, ys_ref, lp_ref, xct_ref, fc_ref,
                segmax_ref, out_ref):
    i = pl.program_id(0)

    @pl.when(i == 0)
    def _():
        out_ref[...] = jnp.zeros(out_ref.shape, jnp.float32)

    f = f_ref[...]
    q = _atanh2(f) + Q_MIN
    qc = _atanh2(fc_ref[...]) + Q_MIN

    dist = jnp.zeros((H_B, N_TRUE), jnp.float32)
    for d in range(3):
        diff = x_ref[:, d:d + 1] - xct_ref[d:d + 1, :]
        dist = dist + diff * diff
    base = jnp.maximum(1.0 - dist, 0.0) * qc
    v_base = jnp.sum(jnp.sum(base, axis=1, keepdims=True) * q)

    bkg = (yi_ref[...] == -1) & (ys_ref[...] >= 0)
    bkgf = bkg.astype(jnp.float32)
    n_bkg = jnp.sum(bkgf)
    bkg_sum = jnp.sum(bkgf * f)
    xi = jnp.where(bkg, 0.0, _atanh2(f))
    xi_sum = jnp.sum(xi)
    lp_xi = jnp.sum(jnp.sum(lp_ref[...], axis=1, keepdims=True) * xi)
    fcs = jnp.where(i == 0,
                    jnp.sum(jnp.maximum(segmax_ref[...], 0.0)), 0.0)

    lane = jax.lax.broadcasted_iota(jnp.int32, (1, 128), 1)
    upd = (jnp.where(lane == 0, v_base, 0.0)
           + jnp.where(lane == 1, n_bkg, 0.0)
           + jnp.where(lane == 2, bkg_sum, 0.0)
           + jnp.where(lane == 3, xi_sum, 0.0)
           + jnp.where(lane == 4, lp_xi, 0.0)
           + jnp.where(lane == 5, fcs, 0.0))
    out_ref[...] = out_ref[...] + upd


def _edge_kernel(ep_ref, xe_ref, fe_ref, w_ref, xct_ref, fc_ref, out_ref):
    i = pl.program_id(0)

    @pl.when(i == 0)
    def _():
        out_ref[...] = jnp.zeros(out_ref.shape, jnp.float32)

    lane = jax.lax.broadcasted_iota(jnp.int32, (E_B, N_TRUE), 1)
    mask = ep_ref[...] == lane
    dist = jnp.zeros((E_B, N_TRUE), jnp.float32)
    for d in range(3):
        diff = xe_ref[:, d:d + 1] - xct_ref[d:d + 1, :]
        dist = dist + diff * diff
    d_e = jnp.sum(jnp.where(mask, dist, 0.0), axis=1, keepdims=True)
    qc = _atanh2(fc_ref[...]) + Q_MIN
    qcg = jnp.sum(jnp.where(mask, qc, 0.0), axis=1, keepdims=True)
    qe = _atanh2(fe_ref[...]) + Q_MIN
    corr = jnp.sum(w_ref[...] * qe * qcg
                   * (d_e - jnp.maximum(1.0 - d_e, 0.0)))
    lane128 = jax.lax.broadcasted_iota(jnp.int32, (1, 128), 1)
    out_ref[...] = out_ref[...] + jnp.where(lane128 == 0, corr, 0.0)


def kernel(x, f, y_i, y_s, n_true, e_true, l_p):
    x = x.astype(jnp.float32)
    f = f.astype(jnp.float32)
    e_h = e_true[0].astype(jnp.int32)
    e_p = e_true[1].astype(jnp.int32)
    vals = f[e_h]

    ep2 = e_p.reshape(-1, 1)
    vals2 = vals.reshape(-1, 1)
    n_eb = N_EDGE // E_B

    segmax = pl.pallas_call(
        _segmax_kernel,
        grid=(n_eb,),
        in_specs=[pl.BlockSpec((E_B, 1), lambda i: (i, 0)),
                  pl.BlockSpec((E_B, 1), lambda i: (i, 0))],
        out_specs=pl.BlockSpec((1, N_TRUE), lambda i: (0, 0)),
        out_shape=jax.ShapeDtypeStruct((1, N_TRUE), jnp.float32),
    )(ep2, vals2)

    arg = pl.pallas_call(
        _argmin_kernel,
        grid=(n_eb,),
        in_specs=[pl.BlockSpec((E_B, 1), lambda i: (i, 0)),
                  pl.BlockSpec((E_B, 1), lambda i: (i, 0)),
                  pl.BlockSpec((1, N_TRUE), lambda i: (0, 0))],
        out_specs=pl.BlockSpec((1, N_TRUE), lambda i: (0, 0)),
        out_shape=jax.ShapeDtypeStruct((1, N_TRUE), jnp.int32),
    )(ep2, vals2, segmax)

    centers = e_h[arg[0]]
    xct = x[centers].T
    fc_row = f[centers].reshape(1, N_TRUE)

    keys = jnp.sort(e_h * N_TRUE + e_p)
    prev = jnp.concatenate([jnp.full((1,), -1, jnp.int32), keys[:-1]])
    w = (keys != prev).astype(jnp.float32).reshape(-1, 1)
    eh_s = keys // N_TRUE
    ep_s = (keys % N_TRUE).reshape(-1, 1)
    xe = x[eh_s]
    fe = f[eh_s].reshape(-1, 1)

    n_hb = N_HIT // H_B
    acc = pl.pallas_call(
        _hit_kernel,
        grid=(n_hb,),
        in_specs=[pl.BlockSpec((H_B, 3), lambda i: (i, 0)),
                  pl.BlockSpec((H_B, 1), lambda i: (i, 0)),
                  pl.BlockSpec((H_B, 1), lambda i: (i, 0)),
                  pl.BlockSpec((H_B, 1), lambda i: (i, 0)),
                  pl.BlockSpec((H_B, 4), lambda i: (i, 0)),
                  pl.BlockSpec((3, N_TRUE), lambda i: (0, 0)),
                  pl.BlockSpec((1, N_TRUE), lambda i: (0, 0)),
                  pl.BlockSpec((1, N_TRUE), lambda i: (0, 0))],
        out_specs=pl.BlockSpec((1, 128), lambda i: (0, 0)),
        out_shape=jax.ShapeDtypeStruct((1, 128), jnp.float32),
    )(x, f.reshape(-1, 1), y_i.reshape(-1, 1).astype(jnp.int32),
      y_s.reshape(-1, 1).astype(jnp.int32), l_p.astype(jnp.float32),
      xct, fc_row, segmax)

    corr = pl.pallas_call(
        _edge_kernel,
        grid=(n_eb,),
        in_specs=[pl.BlockSpec((E_B, 1), lambda i: (i, 0)),
                  pl.BlockSpec((E_B, 3), lambda i: (i, 0)),
                  pl.BlockSpec((E_B, 1), lambda i: (i, 0)),
                  pl.BlockSpec((E_B, 1), lambda i: (i, 0)),
                  pl.BlockSpec((3, N_TRUE), lambda i: (0, 0)),
                  pl.BlockSpec((1, N_TRUE), lambda i: (0, 0))],
        out_specs=pl.BlockSpec((1, 128), lambda i: (0, 0)),
        out_shape=jax.ShapeDtypeStruct((1, 128), jnp.float32),
    )(ep_s, xe, fe, w, xct, fc_row)

    v_base = acc[0, 0]
    n_bkg = acc[0, 1]
    bkg_sum = acc[0, 2]
    xi_sum = acc[0, 3]
    lp_xi = acc[0, 4]
    fc_sum = acc[0, 5]

    b = 1.0 - fc_sum / n_true
    b = b + jnp.where(n_bkg > 0,
                      (S_B / jnp.maximum(n_bkg, 1.0)) * bkg_sum, 0.0)
    v = (v_base + corr[0, 0]) / N_HIT
    p = lp_xi / xi_sum
    return jnp.stack([b, v, p])
